# baseline (speedup 1.0000x reference)
import jax
import jax.numpy as jnp
from jax import lax
from jax.experimental import pallas as pl
from jax.experimental.pallas import tpu as pltpu

THRESH = 0.5
ALPHA = 0.1
_NS = 5


def _start(x_hbm, y_hbm, xb, yb, sx, sy, b):
    s = lax.rem(b, _NS)
    pltpu.make_async_copy(x_hbm.at[b], xb.at[s], sx.at[s]).start()
    pltpu.make_async_copy(y_hbm.at[b], yb.at[s], sy.at[s]).start()


def _body(x_hbm, y_hbm, o_ref, xb, yb, sx, sy):
    B = x_hbm.shape[0]
    o_ref[...] = jnp.zeros_like(o_ref)
    for b in range(_NS):
        _start(x_hbm, y_hbm, xb, yb, sx, sy, jnp.int32(b))

    def step(b, carry):
        s = lax.rem(b, _NS)
        pltpu.make_async_copy(x_hbm.at[b], xb.at[s], sx.at[s]).wait()
        pltpu.make_async_copy(y_hbm.at[b], yb.at[s], sy.at[s]).wait()
        xv = xb[s]
        yv = yb[s]
        mf = (yv[:, 0:1] > THRESH).astype(jnp.float32)
        t = yv - xv
        u = t * mf
        v = xv * mf
        o_ref[0, :] += jnp.sum(u * t, axis=0)
        o_ref[1, :] += jnp.sum(v * xv, axis=0)
        o_ref[2, :] += jnp.sum(xv * xv, axis=0)

        @pl.when(b + _NS < B)
        def _():
            _start(x_hbm, y_hbm, xb, yb, sx, sy, b + _NS)

        return carry

    lax.fori_loop(0, B, step, 0)


def kernel(x, y):
    B, N, C = x.shape
    out = pl.pallas_call(
        _body,
        in_specs=[
            pl.BlockSpec(memory_space=pltpu.HBM),
            pl.BlockSpec(memory_space=pltpu.HBM),
        ],
        out_specs=pl.BlockSpec(memory_space=pltpu.VMEM),
        out_shape=jax.ShapeDtypeStruct((3, C), jnp.float32),
        scratch_shapes=[
            pltpu.VMEM((_NS, N, C), jnp.float32),
            pltpu.VMEM((_NS, N, C), jnp.float32),
            pltpu.SemaphoreType.DMA((_NS,)),
            pltpu.SemaphoreType.DMA((_NS,)),
        ],
    )(x, y)
    return (jnp.sum(out[0]) - ALPHA * jnp.sum(out[1])
            + ALPHA * out[2, 0])
